# baseline (speedup 1.0000x reference)
import functools

import jax
import jax.numpy as jnp
from jax import lax
from jax.experimental import pallas as pl
from jax.experimental.pallas import tpu as pltpu
from jax.experimental.pallas import tpu_sc as plsc

GROUPS = 2
NUM_VARS = 320
VAR_DIM = 128

BM = 1024
SB = 256


def _tc_body(x_ref, w1_ref, b1_ref, w2_hbm, b2_ref,
             idx_ref, code_ppl_ref, prob_ppl_ref,
             acc0, acc1, cnt0, cnt1, w2_vmem, w2_sem):
    i = pl.program_id(0)
    n = pl.num_programs(0)
    accs = (acc0, acc1)
    cnts = (cnt0, cnt1)

    @pl.when(i == 0)
    def _init():
        for r in accs + cnts:
            r[...] = jnp.zeros_like(r)
        pltpu.make_async_copy(w2_hbm, w2_vmem, w2_sem).start()

    iota = lax.broadcasted_iota(jnp.int32, (SB, NUM_VARS), 1)
    for t in range(BM // SB):
        rows = pl.ds(t * SB, SB)
        h = jnp.dot(x_ref[rows, :], w1_ref[...],
                    preferred_element_type=jnp.float32)
        h = h + b1_ref[...]
        h = h * 0.5 * (lax.erf(h * 0.7071067811865476) + 1.0)
        if t == 0:
            @pl.when(i == 0)
            def _wait_w2():
                pltpu.make_async_copy(w2_hbm, w2_vmem, w2_sem).wait()
        logits = jnp.dot(h, w2_vmem[...], preferred_element_type=jnp.float32)
        logits = logits + b2_ref[...]

        idx_ref[rows, :] = (logits[:, :GROUPS] > 0).astype(jnp.int32)

    @pl.when(i == n - 1)
    def _finish():
        nt = jnp.float32(n * BM)

        def ppl(rows):
            t = 0.0
            for r in rows:
                p = r[...] / nt
                t += jnp.exp(-jnp.sum(p * jnp.log(p + 1e-7)))
            return t

        code_ppl_ref[...] = jnp.reshape(ppl(cnts), (1, 1))
        prob_ppl_ref[...] = jnp.reshape(ppl(accs), (1, 1))


def _tc_forward(xf, W1, b1, W2, b2):
    nt = xf.shape[0]
    grid = (nt // BM,)
    return pl.pallas_call(
        _tc_body,
        grid=grid,
        in_specs=[
            pl.BlockSpec((BM, xf.shape[1]), lambda i: (i, 0)),
            pl.BlockSpec(W1.shape, lambda i: (0, 0)),
            pl.BlockSpec((1, b1.shape[1]), lambda i: (0, 0)),
            pl.BlockSpec(memory_space=pl.ANY),
            pl.BlockSpec((1, b2.shape[1]), lambda i: (0, 0)),
        ],
        out_specs=[
            pl.BlockSpec((BM, GROUPS), lambda i: (i, 0)),
            pl.BlockSpec((1, 1), lambda i: (0, 0)),
            pl.BlockSpec((1, 1), lambda i: (0, 0)),
        ],
        out_shape=[
            jax.ShapeDtypeStruct((nt, GROUPS), jnp.int32),
            jax.ShapeDtypeStruct((1, 1), jnp.float32),
            jax.ShapeDtypeStruct((1, 1), jnp.float32),
        ],
        scratch_shapes=[pltpu.VMEM((1, NUM_VARS), jnp.float32)] * 4
        + [pltpu.VMEM(W2.shape, jnp.float32), pltpu.SemaphoreType.DMA],
    )(xf, W1, b1, W2, b2)


def _sc_gather(table, idx3):
    info = plsc.get_sparse_core_info()
    nw = info.num_cores * info.num_subcores
    nrows = idx3.shape[0] * idx3.shape[1] * idx3.shape[2]
    ch, kg = idx3.shape[1], idx3.shape[2]
    b_per_w = ch * kg
    mesh = plsc.VectorSubcoreMesh(core_axis_name="c", subcore_axis_name="s")

    @functools.partial(
        pl.kernel,
        mesh=mesh,
        out_type=jax.ShapeDtypeStruct((nrows, VAR_DIM), jnp.float32),
        scratch_types=[
            pltpu.VMEM((ch, kg), jnp.int32),
            pltpu.VMEM((b_per_w, VAR_DIM), jnp.float32),
            pltpu.SemaphoreType.DMA,
        ],
    )
    def k(table_hbm, idx_hbm, out_hbm, idx_v, rows_v, sem):
        wid = lax.axis_index("s") * info.num_cores + lax.axis_index("c")
        pltpu.sync_copy(idx_hbm.at[wid], idx_v)
        copies = [
            pltpu.async_copy(table_hbm.at[idx_v.at[j]],
                             rows_v.at[pl.ds(j * kg, kg)], sem)
            for j in range(ch)
        ]
        for c in copies:
            c.wait()
        pltpu.sync_copy(rows_v, out_hbm.at[pl.ds(wid * b_per_w, b_per_w)])

    return k(table, idx3)


def kernel(x, codebook, W1, b1, W2, b2):
    bsz, tsz, fsz = x.shape
    xf = x.reshape(bsz * tsz, fsz)
    idx, code_ppl, prob_ppl = _tc_forward(
        xf, W1, b1.reshape(1, -1), W2, b2.reshape(1, -1))
    table = codebook.reshape(GROUPS * NUM_VARS, VAR_DIM)
    idx3 = idx.reshape(32, -1, 128)
    gathered = _sc_gather(table, idx3)
    out = gathered.reshape(bsz, tsz, GROUPS * VAR_DIM)
    return out, code_ppl.reshape(()), prob_ppl.reshape(())


# speedup vs baseline: 2.6905x; 2.6905x over previous
import functools

import jax
import jax.numpy as jnp
from jax import lax
from jax.experimental import pallas as pl
from jax.experimental.pallas import tpu as pltpu
from jax.experimental.pallas import tpu_sc as plsc

GROUPS = 2
NUM_VARS = 320
VAR_DIM = 128

BM = 1024
SB = 256


def _tc_body(x_ref, w1_ref, b1_ref, w2_ref, b2_ref,
             idx_ref, code_ppl_ref, prob_ppl_ref,
             acc0, acc1, cnt0, cnt1):
    i = pl.program_id(0)
    n = pl.num_programs(0)
    accs = (acc0, acc1)
    cnts = (cnt0, cnt1)

    @pl.when(i == 0)
    def _init():
        for r in accs + cnts:
            r[...] = jnp.zeros_like(r)

    iota = lax.broadcasted_iota(jnp.int32, (SB, NUM_VARS), 1)
    for t in range(BM // SB):
        rows = pl.ds(t * SB, SB)
        h = jnp.dot(x_ref[rows, :], w1_ref[...],
                    preferred_element_type=jnp.float32)
        h = h + b1_ref[...]
        h = h * 0.5 * (lax.erf(h * 0.7071067811865476) + 1.0)
        logits = jnp.dot(h, w2_ref[...], preferred_element_type=jnp.float32)
        logits = logits + b2_ref[...]

        ks = []
        for g in range(GROUPS):
            lg = logits[:, g * NUM_VARS:(g + 1) * NUM_VARS]
            m = jnp.max(lg, axis=1, keepdims=True)
            kg = jnp.min(jnp.where(lg == m, iota, NUM_VARS), axis=1,
                         keepdims=True)
            e = jnp.exp(lg - m)
            s = jnp.sum(e, axis=1, keepdims=True)
            accs[g][...] += jnp.sum(e / s, axis=0, keepdims=True)
            cnts[g][...] += jnp.sum((iota == kg).astype(jnp.float32), axis=0,
                                    keepdims=True)
            ks.append(kg + g * NUM_VARS)
        idx_ref[rows, :] = jnp.concatenate(ks, axis=1)

    @pl.when(i == n - 1)
    def _finish():
        nt = jnp.float32(n * BM)

        def ppl(rows):
            t = 0.0
            for r in rows:
                p = r[...] / nt
                t += jnp.exp(-jnp.sum(p * jnp.log(p + 1e-7)))
            return t

        code_ppl_ref[...] = jnp.reshape(ppl(cnts), (1, 1))
        prob_ppl_ref[...] = jnp.reshape(ppl(accs), (1, 1))


def _tc_forward(xf, W1, b1, W2, b2):
    nt = xf.shape[0]
    grid = (nt // BM,)
    return pl.pallas_call(
        _tc_body,
        grid=grid,
        in_specs=[
            pl.BlockSpec((BM, xf.shape[1]), lambda i: (i, 0)),
            pl.BlockSpec(W1.shape, lambda i: (0, 0)),
            pl.BlockSpec((1, b1.shape[1]), lambda i: (0, 0)),
            pl.BlockSpec(W2.shape, lambda i: (0, 0)),
            pl.BlockSpec((1, b2.shape[1]), lambda i: (0, 0)),
        ],
        out_specs=[
            pl.BlockSpec((BM, GROUPS), lambda i: (i, 0)),
            pl.BlockSpec((1, 1), lambda i: (0, 0)),
            pl.BlockSpec((1, 1), lambda i: (0, 0)),
        ],
        out_shape=[
            jax.ShapeDtypeStruct((nt, GROUPS), jnp.int32),
            jax.ShapeDtypeStruct((1, 1), jnp.float32),
            jax.ShapeDtypeStruct((1, 1), jnp.float32),
        ],
        scratch_shapes=[pltpu.VMEM((1, NUM_VARS), jnp.float32)] * 4,
    )(xf, W1, b1, W2, b2)


def _sc_gather(table, idx3):
    info = plsc.get_sparse_core_info()
    nw = info.num_cores * info.num_subcores
    nrows = idx3.shape[0] * idx3.shape[1] * idx3.shape[2]
    ch, kg = idx3.shape[1], idx3.shape[2]
    b_per_w = ch * kg
    mesh = plsc.VectorSubcoreMesh(core_axis_name="c", subcore_axis_name="s")

    @functools.partial(
        pl.kernel,
        mesh=mesh,
        out_type=jax.ShapeDtypeStruct((nrows, VAR_DIM), jnp.float32),
        scratch_types=[
            pltpu.VMEM((ch, kg), jnp.int32),
            pltpu.VMEM((b_per_w, VAR_DIM), jnp.float32),
            pltpu.SemaphoreType.DMA,
        ],
    )
    def k(table_hbm, idx_hbm, out_hbm, idx_v, rows_v, sem):
        wid = lax.axis_index("s") * info.num_cores + lax.axis_index("c")
        pltpu.sync_copy(idx_hbm.at[wid], idx_v)
        copies = [
            pltpu.async_copy(table_hbm.at[idx_v.at[j]],
                             rows_v.at[pl.ds(j * kg, kg)], sem)
            for j in range(ch)
        ]
        for c in copies:
            c.wait()
        pltpu.sync_copy(rows_v, out_hbm.at[pl.ds(wid * b_per_w, b_per_w)])

    return k(table, idx3)


def kernel(x, codebook, W1, b1, W2, b2):
    bsz, tsz, fsz = x.shape
    xf = x.reshape(bsz * tsz, fsz)
    idx, code_ppl, prob_ppl = _tc_forward(
        xf, W1, b1.reshape(1, -1), W2, b2.reshape(1, -1))
    table = codebook.reshape(GROUPS * NUM_VARS, VAR_DIM)
    idx3 = idx.reshape(32, -1, 128)
    gathered = _sc_gather(table, idx3)
    out = gathered.reshape(bsz, tsz, GROUPS * VAR_DIM)
    return out, code_ppl.reshape(()), prob_ppl.reshape(())
